# Initial kernel scaffold; baseline (speedup 1.0000x reference)
import jax
import jax.numpy as jnp
from jax.experimental import pallas as pl


def kernel(batch_images, representations, actions):
    raise NotImplementedError("write your pallas kernel here")


# speedup vs baseline: 4.4153x; 4.4153x over previous
import functools

import jax
import jax.numpy as jnp
from jax import lax
from jax.experimental import pallas as pl
from jax.experimental.pallas import tpu as pltpu
from jax.experimental.pallas import tpu_sc as plsc

Q = 1024
D = 128
N = 100000
KNB = 8
ADIM = 7
L = 2048
NBLK = 49
NPAD = L * NBLK
RT = 256
NROWT = Q // RT
RTE = 64
NEXT = Q // RTE


def _stream_body(q_ref, a2_ref, k_ref, b2_ref,
                 m1, m2, m3, i1, i2, i3):
    j = pl.program_id(0)
    i = pl.program_id(1)

    @pl.when(j == 0)
    def _init():
        inf = jnp.full((RT, L), jnp.inf, jnp.float32)
        zero = jnp.zeros((RT, L), jnp.int32)
        m1[i] = inf
        m2[i] = inf
        m3[i] = inf
        i1[i] = zero
        i2[i] = zero
        i3[i] = zero

    qv = q_ref[i]
    kv = k_ref[...]
    ab = lax.dot_general(qv, kv, (((1,), (1,)), ((), ())),
                         preferred_element_type=jnp.float32)
    f = (a2_ref[i] + b2_ref[...]) - 2.0 * ab
    idx = j * L + lax.broadcasted_iota(jnp.int32, (RT, L), 1)

    om1 = m1[i]
    om2 = m2[i]
    om3 = m3[i]
    oi1 = i1[i]
    oi2 = i2[i]
    c1 = f < om1
    c2 = f < om2
    c3 = f < om3
    m1[i] = jnp.where(c1, f, om1)
    i1[i] = jnp.where(c1, idx, oi1)
    m2[i] = jnp.where(c1, om1, jnp.where(c2, f, om2))
    i2[i] = jnp.where(c1, oi1, jnp.where(c2, idx, oi2))
    m3[i] = jnp.where(c2, om2, jnp.where(c3, f, om3))
    i3[i] = jnp.where(c2, oi2, jnp.where(c3, idx, i3[i]))


_stream_call = pl.pallas_call(
    _stream_body,
    grid=(NBLK, NROWT),
    in_specs=[
        pl.BlockSpec((NROWT, RT, D), lambda j, i: (0, 0, 0)),
        pl.BlockSpec((NROWT, RT, 1), lambda j, i: (0, 0, 0)),
        pl.BlockSpec((L, D), lambda j, i: (j, 0)),
        pl.BlockSpec((1, L), lambda j, i: (0, j)),
    ],
    out_specs=[pl.BlockSpec((NROWT, RT, L), lambda j, i: (0, 0, 0))
               for _ in range(6)],
    out_shape=[jax.ShapeDtypeStruct((NROWT, RT, L), jnp.float32)
               for _ in range(3)] +
              [jax.ShapeDtypeStruct((NROWT, RT, L), jnp.int32)
               for _ in range(3)],
    compiler_params=pltpu.CompilerParams(
        dimension_semantics=("arbitrary", "arbitrary"),
    ),
)


def _extract_body(m1_ref, m2_ref, m3_ref, i1_ref, i2_ref, i3_ref,
                  ti_ref, w_ref):
    d1 = jnp.sqrt(jnp.maximum(m1_ref[0], 1e-12))
    d2 = jnp.sqrt(jnp.maximum(m2_ref[0], 1e-12))
    d3 = jnp.sqrt(jnp.maximum(m3_ref[0], 1e-12))
    j1 = i1_ref[0]
    j2 = i2_ref[0]
    j3 = i3_ref[0]
    bigi = jnp.int32(2**30)
    inf = jnp.float32(jnp.inf)
    top_d = []
    top_i = []
    for _ in range(KNB):
        rm = jnp.min(d1, axis=1, keepdims=True)
        si = jnp.min(jnp.where(d1 == rm, j1, bigi), axis=1, keepdims=True)
        sel = (d1 == rm) & (j1 == si)
        top_d.append(rm)
        top_i.append(si)
        d1 = jnp.where(sel, d2, d1)
        j1 = jnp.where(sel, j2, j1)
        d2 = jnp.where(sel, d3, d2)
        j2 = jnp.where(sel, j3, j2)
        d3 = jnp.where(sel, inf, d3)
    td = jnp.concatenate(top_d, axis=1)
    ti = jnp.concatenate(top_i, axis=1)
    e = jnp.exp(td[:, 0:1] - td)
    w = e / jnp.sum(e, axis=1, keepdims=True)
    ti_ref[...] = ti
    w_ref[...] = w


_extract_call = pl.pallas_call(
    _extract_body,
    grid=(NEXT,),
    in_specs=[pl.BlockSpec((1, RTE, L), lambda e: (e, 0, 0))
              for _ in range(6)],
    out_specs=[
        pl.BlockSpec((RTE, KNB), lambda e: (e, 0)),
        pl.BlockSpec((RTE, KNB), lambda e: (e, 0)),
    ],
    out_shape=[
        jax.ShapeDtypeStruct((Q, KNB), jnp.int32),
        jax.ShapeDtypeStruct((Q, KNB), jnp.float32),
    ],
    compiler_params=pltpu.CompilerParams(
        dimension_semantics=("arbitrary",),
    ),
)


NW = 32
BPW = Q * KNB // NW
QPW = Q // NW
GCH = 4
GCW = BPW // GCH


def _sc_blend_body(idx_hbm, w_hbm, act_hbm, out_hbm, idx_v, w_v, rows_v,
                   acc_v, sem):
    wid = lax.axis_index("s") * 2 + lax.axis_index("c")
    pltpu.sync_copy(idx_hbm.at[wid], idx_v)
    pltpu.sync_copy(w_hbm.at[wid], w_v)
    copies = [
        pltpu.async_copy(act_hbm.at[idx_v.at[c]],
                         rows_v.at[pl.ds(c * GCW, GCW)], sem)
        for c in range(GCH)
    ]
    for cp in copies:
        cp.wait()
    for qq in range(QPW):
        acc = w_v[qq * KNB, :] * rows_v[qq * KNB, :]
        for k in range(1, KNB):
            acc = acc + w_v[qq * KNB + k, :] * rows_v[qq * KNB + k, :]
        acc_v[qq, :] = acc
    pltpu.sync_copy(acc_v, out_hbm.at[pl.ds(wid * QPW, QPW)])


_sc_blend_call = functools.partial(
    pl.kernel,
    out_type=jax.ShapeDtypeStruct((Q, 16), jnp.float32),
    mesh=plsc.VectorSubcoreMesh(core_axis_name="c", subcore_axis_name="s"),
    scratch_types=[
        pltpu.VMEM((GCH, GCW), jnp.int32),
        pltpu.VMEM((BPW, 16), jnp.float32),
        pltpu.VMEM((BPW, 16), jnp.float32),
        pltpu.VMEM((QPW, 16), jnp.float32),
        pltpu.SemaphoreType.DMA,
    ],
    compiler_params=pltpu.CompilerParams(use_tc_tiling_on_sc=False),
)(_sc_blend_body)


def kernel(batch_images, representations, actions):
    qm = jnp.squeeze(batch_images, axis=1)
    a2 = jnp.sum(qm * qm, axis=1, keepdims=True)
    b2 = jnp.sum(representations * representations, axis=1)
    kpad = jnp.pad(representations, ((0, NPAD - N), (0, 0)))
    b2pad = jnp.pad(b2, (0, NPAD - N), constant_values=jnp.inf)[None, :]
    q3 = qm.reshape(NROWT, RT, D)
    a23 = a2.reshape(NROWT, RT, 1)
    m1, m2, m3, i1, i2, i3 = _stream_call(q3, a23, kpad, b2pad)

    rs = lambda x: x.reshape(NEXT, RTE, L)
    ti, w = _extract_call(rs(m1), rs(m2), rs(m3), rs(i1), rs(i2), rs(i3))

    apad = jnp.pad(actions, ((0, 0), (0, 16 - ADIM)))
    idx_g = ti.reshape(NW, GCH, GCW)
    w_g = jnp.repeat(w.reshape(Q * KNB, 1), 16, axis=1).reshape(NW, BPW, 16)
    pred_pad = _sc_blend_call(idx_g, w_g, apad)
    return pred_pad[:, :ADIM], ti


# speedup vs baseline: 4.7207x; 1.0692x over previous
import functools

import jax
import jax.numpy as jnp
from jax import lax
from jax.experimental import pallas as pl
from jax.experimental.pallas import tpu as pltpu
from jax.experimental.pallas import tpu_sc as plsc

Q = 1024
D = 128
N = 100000
KNB = 8
ADIM = 7
L = 2048
BH = 2
B = L * BH
NBLK = 25
NPADB = B * NBLK
RT = 256
NROWT = Q // RT
RTE = 64
NEXT = Q // RTE


def _stream_body(q_ref, a2_ref, k_ref, b2_ref,
                 m1, m2, m3, i1, i2, i3):
    j = pl.program_id(1)

    @pl.when(j == 0)
    def _init():
        inf = jnp.full((RT, L), jnp.inf, jnp.float32)
        zero = jnp.zeros((RT, L), jnp.int32)
        m1[0] = inf
        m2[0] = inf
        m3[0] = inf
        i1[0] = zero
        i2[0] = zero
        i3[0] = zero

    qv = q_ref[0]
    kv = k_ref[...]
    ab = lax.dot_general(qv, kv, (((1,), (1,)), ((), ())),
                         preferred_element_type=jnp.float32)
    f = (a2_ref[0] + b2_ref[...]) - 2.0 * ab

    om1 = m1[0]
    om2 = m2[0]
    om3 = m3[0]
    oi1 = i1[0]
    oi2 = i2[0]
    oi3 = i3[0]
    for h in range(BH):
        x = f[:, h * L:(h + 1) * L]
        p = j * BH + h
        c1 = x < om1
        c2 = x < om2
        c3 = x < om3
        nm1 = jnp.where(c1, x, om1)
        ni1 = jnp.where(c1, p, oi1)
        nm2 = jnp.where(c1, om1, jnp.where(c2, x, om2))
        ni2 = jnp.where(c1, oi1, jnp.where(c2, p, oi2))
        nm3 = jnp.where(c2, om2, jnp.where(c3, x, om3))
        ni3 = jnp.where(c2, oi2, jnp.where(c3, p, oi3))
        om1, om2, om3 = nm1, nm2, nm3
        oi1, oi2, oi3 = ni1, ni2, ni3
    m1[0] = om1
    m2[0] = om2
    m3[0] = om3
    i1[0] = oi1
    i2[0] = oi2
    i3[0] = oi3


_stream_call = pl.pallas_call(
    _stream_body,
    grid=(NROWT, NBLK),
    in_specs=[
        pl.BlockSpec((1, RT, D), lambda i, j: (i, 0, 0)),
        pl.BlockSpec((1, RT, 1), lambda i, j: (i, 0, 0)),
        pl.BlockSpec((B, D), lambda i, j: (j, 0)),
        pl.BlockSpec((1, B), lambda i, j: (0, j)),
    ],
    out_specs=[pl.BlockSpec((1, RT, L), lambda i, j: (i, 0, 0))
               for _ in range(6)],
    out_shape=[jax.ShapeDtypeStruct((NROWT, RT, L), jnp.float32)
               for _ in range(3)] +
              [jax.ShapeDtypeStruct((NROWT, RT, L), jnp.int32)
               for _ in range(3)],
    compiler_params=pltpu.CompilerParams(
        dimension_semantics=("arbitrary", "arbitrary"),
    ),
)


def _extract_body(m1_ref, m2_ref, m3_ref, i1_ref, i2_ref, i3_ref,
                  ti_ref, w_ref):
    d1 = jnp.sqrt(jnp.maximum(m1_ref[0], 1e-12))
    d2 = jnp.sqrt(jnp.maximum(m2_ref[0], 1e-12))
    d3 = jnp.sqrt(jnp.maximum(m3_ref[0], 1e-12))
    lane = lax.broadcasted_iota(jnp.int32, (RTE, L), 1)
    j1 = i1_ref[0] * L + lane
    j2 = i2_ref[0] * L + lane
    j3 = i3_ref[0] * L + lane
    bigi = jnp.int32(2**30)
    inf = jnp.float32(jnp.inf)
    top_d = []
    top_i = []
    for _ in range(KNB):
        rm = jnp.min(d1, axis=1, keepdims=True)
        si = jnp.min(jnp.where(d1 == rm, j1, bigi), axis=1, keepdims=True)
        sel = (d1 == rm) & (j1 == si)
        top_d.append(rm)
        top_i.append(si)
        d1 = jnp.where(sel, d2, d1)
        j1 = jnp.where(sel, j2, j1)
        d2 = jnp.where(sel, d3, d2)
        j2 = jnp.where(sel, j3, j2)
        d3 = jnp.where(sel, inf, d3)
    td = jnp.concatenate(top_d, axis=1)
    ti = jnp.concatenate(top_i, axis=1)
    e = jnp.exp(td[:, 0:1] - td)
    w = e / jnp.sum(e, axis=1, keepdims=True)
    ti_ref[...] = ti
    w_ref[...] = w


_extract_call = pl.pallas_call(
    _extract_body,
    grid=(NEXT,),
    in_specs=[pl.BlockSpec((1, RTE, L), lambda e: (e, 0, 0))
              for _ in range(6)],
    out_specs=[
        pl.BlockSpec((RTE, KNB), lambda e: (e, 0)),
        pl.BlockSpec((RTE, KNB), lambda e: (e, 0)),
    ],
    out_shape=[
        jax.ShapeDtypeStruct((Q, KNB), jnp.int32),
        jax.ShapeDtypeStruct((Q, KNB), jnp.float32),
    ],
    compiler_params=pltpu.CompilerParams(
        dimension_semantics=("arbitrary",),
    ),
)


NW = 32
BPW = Q * KNB // NW
QPW = Q // NW
GCH = 4
GCW = BPW // GCH


def _sc_blend_body(idx_hbm, w_hbm, act_hbm, out_hbm, idx_v, w_v, rows_v,
                   acc_v, sem):
    wid = lax.axis_index("s") * 2 + lax.axis_index("c")
    pltpu.sync_copy(idx_hbm.at[wid], idx_v)
    pltpu.sync_copy(w_hbm.at[wid], w_v)
    copies = [
        pltpu.async_copy(act_hbm.at[idx_v.at[c]],
                         rows_v.at[pl.ds(c * GCW, GCW)], sem)
        for c in range(GCH)
    ]
    for cp in copies:
        cp.wait()
    for qq in range(QPW):
        acc = w_v[qq * KNB, :] * rows_v[qq * KNB, :]
        for k in range(1, KNB):
            acc = acc + w_v[qq * KNB + k, :] * rows_v[qq * KNB + k, :]
        acc_v[qq, :] = acc
    pltpu.sync_copy(acc_v, out_hbm.at[pl.ds(wid * QPW, QPW)])


_sc_blend_call = functools.partial(
    pl.kernel,
    out_type=jax.ShapeDtypeStruct((Q, 16), jnp.float32),
    mesh=plsc.VectorSubcoreMesh(core_axis_name="c", subcore_axis_name="s"),
    scratch_types=[
        pltpu.VMEM((GCH, GCW), jnp.int32),
        pltpu.VMEM((BPW, 16), jnp.float32),
        pltpu.VMEM((BPW, 16), jnp.float32),
        pltpu.VMEM((QPW, 16), jnp.float32),
        pltpu.SemaphoreType.DMA,
    ],
    compiler_params=pltpu.CompilerParams(use_tc_tiling_on_sc=False),
)(_sc_blend_body)


def kernel(batch_images, representations, actions):
    qm = jnp.squeeze(batch_images, axis=1)
    a2 = jnp.sum(qm * qm, axis=1, keepdims=True)
    b2 = jnp.sum(representations * representations, axis=1)
    b2pad = jnp.pad(b2, (0, NPADB - N), constant_values=jnp.inf)[None, :]
    q3 = qm.reshape(NROWT, RT, D)
    a23 = a2.reshape(NROWT, RT, 1)
    m1, m2, m3, i1, i2, i3 = _stream_call(q3, a23, representations, b2pad)

    rs = lambda x: x.reshape(NEXT, RTE, L)
    ti, w = _extract_call(rs(m1), rs(m2), rs(m3), rs(i1), rs(i2), rs(i3))

    apad = jnp.pad(actions, ((0, 0), (0, 16 - ADIM)))
    idx_g = ti.reshape(NW, GCH, GCW)
    w_g = jnp.repeat(w.reshape(Q * KNB, 1), 16, axis=1).reshape(NW, BPW, 16)
    pred_pad = _sc_blend_call(idx_g, w_g, apad)
    return pred_pad[:, :ADIM], ti


# speedup vs baseline: 4.9631x; 1.0513x over previous
import functools

import jax
import jax.numpy as jnp
from jax import lax
from jax.experimental import pallas as pl
from jax.experimental.pallas import tpu as pltpu
from jax.experimental.pallas import tpu_sc as plsc

Q = 1024
D = 128
N = 100000
KNB = 8
ADIM = 7
L = 2048
BH = 2
B = L * BH
NBLK = 25
NPADB = B * NBLK
RT = 256
NROWT = Q // RT
RTE = 64
NEXT = Q // RTE


def _stream_body(q_ref, a2_ref, k_ref, b2_ref,
                 m1, m2, m3, i1, i2, i3):
    j = pl.program_id(1)

    @pl.when(j == 0)
    def _init():
        inf = jnp.full((RT, L), jnp.inf, jnp.float32)
        zero = jnp.zeros((RT, L), jnp.int32)
        m1[0] = inf
        m2[0] = inf
        m3[0] = inf
        i1[0] = zero
        i2[0] = zero
        i3[0] = zero

    qv = q_ref[0]
    kv = k_ref[...]
    ab = lax.dot_general(qv, kv, (((1,), (1,)), ((), ())),
                         preferred_element_type=jnp.float32)
    f = (a2_ref[0] + b2_ref[...]) - 2.0 * ab

    om1 = m1[0]
    om2 = m2[0]
    om3 = m3[0]
    oi1 = i1[0]
    oi2 = i2[0]
    oi3 = i3[0]
    for h in range(BH):
        x = f[:, h * L:(h + 1) * L]
        p = j * BH + h
        c1 = x < om1
        c2 = x < om2
        c3 = x < om3
        nm1 = jnp.where(c1, x, om1)
        ni1 = jnp.where(c1, p, oi1)
        nm2 = jnp.where(c1, om1, jnp.where(c2, x, om2))
        ni2 = jnp.where(c1, oi1, jnp.where(c2, p, oi2))
        nm3 = jnp.where(c2, om2, jnp.where(c3, x, om3))
        ni3 = jnp.where(c2, oi2, jnp.where(c3, p, oi3))
        om1, om2, om3 = nm1, nm2, nm3
        oi1, oi2, oi3 = ni1, ni2, ni3
    m1[0] = om1
    m2[0] = om2
    m3[0] = om3
    i1[0] = oi1
    i2[0] = oi2
    i3[0] = oi3


_stream_call = pl.pallas_call(
    _stream_body,
    grid=(NROWT, NBLK),
    in_specs=[
        pl.BlockSpec((1, RT, D), lambda i, j: (i, 0, 0)),
        pl.BlockSpec((1, RT, 1), lambda i, j: (i, 0, 0)),
        pl.BlockSpec((B, D), lambda i, j: (j, 0)),
        pl.BlockSpec((1, B), lambda i, j: (0, j)),
    ],
    out_specs=[pl.BlockSpec((1, RT, L), lambda i, j: (i, 0, 0))
               for _ in range(6)],
    out_shape=[jax.ShapeDtypeStruct((NROWT, RT, L), jnp.float32)
               for _ in range(3)] +
              [jax.ShapeDtypeStruct((NROWT, RT, L), jnp.int32)
               for _ in range(3)],
    compiler_params=pltpu.CompilerParams(
        dimension_semantics=("arbitrary", "arbitrary"),
    ),
)


def _extract_body(m1_ref, m2_ref, m3_ref, i1_ref, i2_ref, i3_ref,
                  ti_ref, w_ref):
    d1 = jnp.sqrt(jnp.maximum(m1_ref[0], 1e-12))
    d2 = jnp.sqrt(jnp.maximum(m2_ref[0], 1e-12))
    d3 = jnp.sqrt(jnp.maximum(m3_ref[0], 1e-12))
    lane = lax.broadcasted_iota(jnp.int32, (RTE, L), 1)
    j1 = i1_ref[0] * L + lane
    j2 = i2_ref[0] * L + lane
    j3 = i3_ref[0] * L + lane
    bigi = jnp.int32(2**30)
    inf = jnp.float32(jnp.inf)
    top_d = []
    top_i = []
    for _ in range(KNB):
        rm = jnp.min(d1, axis=1, keepdims=True)
        si = jnp.min(jnp.where(d1 == rm, j1, bigi), axis=1, keepdims=True)
        sel = (d1 == rm) & (j1 == si)
        top_d.append(rm)
        top_i.append(si)
        d1 = jnp.where(sel, d2, d1)
        j1 = jnp.where(sel, j2, j1)
        d2 = jnp.where(sel, d3, d2)
        j2 = jnp.where(sel, j3, j2)
        d3 = jnp.where(sel, inf, d3)
    td = jnp.concatenate(top_d, axis=1)
    ti = jnp.concatenate(top_i, axis=1)
    e = jnp.exp(td[:, 0:1] - td)
    w = e / jnp.sum(e, axis=1, keepdims=True)
    ti_ref[...] = ti
    w_ref[...] = w


_extract_call = pl.pallas_call(
    _extract_body,
    grid=(NEXT,),
    in_specs=[pl.BlockSpec((1, RTE, L), lambda e: (e // (RT // RTE), e % (RT // RTE), 0))
              for _ in range(6)],
    out_specs=[
        pl.BlockSpec((RTE, KNB), lambda e: (e, 0)),
        pl.BlockSpec((RTE, KNB), lambda e: (e, 0)),
    ],
    out_shape=[
        jax.ShapeDtypeStruct((Q, KNB), jnp.int32),
        jax.ShapeDtypeStruct((Q, KNB), jnp.float32),
    ],
    compiler_params=pltpu.CompilerParams(
        dimension_semantics=("arbitrary",),
    ),
)


NW = 32
QPW = Q // NW
WPW = QPW * KNB * 8
GCH = 16
GCW = WPW // GCH
NV = QPW * 8 // 16


def _sc_blend_body(idx_hbm, w_hbm, act_hbm, out_hbm, idx_v, w_v, rows_v,
                   acc_v, sem):
    wid = lax.axis_index("s") * 2 + lax.axis_index("c")
    pltpu.sync_copy(idx_hbm.at[wid], idx_v)
    pltpu.sync_copy(w_hbm.at[wid], w_v)
    copies = [
        pltpu.async_copy(act_hbm.at[idx_v.at[c]],
                         rows_v.at[pl.ds(c * GCW, GCW)], sem)
        for c in range(GCH)
    ]
    for cp in copies:
        cp.wait()
    for v in range(NV):
        acc = w_v[0, pl.ds(v * 16, 16)] * rows_v[pl.ds(v * 16, 16)]
        for k in range(1, KNB):
            acc = acc + (w_v[k, pl.ds(v * 16, 16)] *
                         rows_v[pl.ds(k * QPW * 8 + v * 16, 16)])
        acc_v[pl.ds(v * 16, 16)] = acc
    pltpu.sync_copy(acc_v, out_hbm.at[wid])


_sc_blend_call = functools.partial(
    pl.kernel,
    out_type=jax.ShapeDtypeStruct((NW, QPW * 8), jnp.float32),
    mesh=plsc.VectorSubcoreMesh(core_axis_name="c", subcore_axis_name="s"),
    scratch_types=[
        pltpu.VMEM((GCH, GCW), jnp.int32),
        pltpu.VMEM((KNB, QPW * 8), jnp.float32),
        pltpu.VMEM((WPW,), jnp.float32),
        pltpu.VMEM((QPW * 8,), jnp.float32),
        pltpu.SemaphoreType.DMA,
    ],
    compiler_params=pltpu.CompilerParams(use_tc_tiling_on_sc=False),
)(_sc_blend_body)


def kernel(batch_images, representations, actions):
    qm = jnp.squeeze(batch_images, axis=1)
    a2 = jnp.sum(qm * qm, axis=1, keepdims=True)
    b2 = jnp.sum(representations * representations, axis=1)
    b2pad = jnp.pad(b2, (0, NPADB - N), constant_values=jnp.inf)[None, :]
    q3 = qm.reshape(NROWT, RT, D)
    a23 = a2.reshape(NROWT, RT, 1)
    m1, m2, m3, i1, i2, i3 = _stream_call(q3, a23, representations, b2pad)

    ti, w = _extract_call(m1, m2, m3, i1, i2, i3)

    af = actions.reshape(N * ADIM)
    toff = jnp.minimum(jnp.arange(8, dtype=jnp.int32), ADIM - 1)
    ie = (ADIM * ti)[:, :, None] + toff
    idx_g = (ie.reshape(NW, QPW, KNB, 8)
             .transpose(0, 2, 1, 3).reshape(NW, GCH, GCW))
    w_g = jnp.broadcast_to(
        w.reshape(NW, QPW, KNB).transpose(0, 2, 1)[..., None],
        (NW, KNB, QPW, 8)).reshape(NW, KNB, QPW * 8)
    pred_w = _sc_blend_call(idx_g, w_g, af)
    return pred_w.reshape(Q, 8)[:, :ADIM], ti


# speedup vs baseline: 5.6738x; 1.1432x over previous
import functools

import jax
import jax.numpy as jnp
from jax import lax
from jax.experimental import pallas as pl
from jax.experimental.pallas import tpu as pltpu
from jax.experimental.pallas import tpu_sc as plsc

Q = 1024
D = 128
N = 100000
KNB = 8
ADIM = 7
L = 2048
BH = 2
B = L * BH
NBLK = 25
NPADB = B * NBLK
RT = 256
NROWT = Q // RT
RTE = 64
NEXT = Q // RTE


def _stream_body(q_ref, a2_ref, k_ref, b2_ref,
                 m1, m2, m3, i1, i2, i3):
    j = pl.program_id(1)

    @pl.when(j == 0)
    def _init():
        inf = jnp.full((RT, L), jnp.inf, jnp.float32)
        zero = jnp.zeros((RT, L), jnp.int32)
        m1[0] = inf
        m2[0] = inf
        m3[0] = inf
        i1[0] = zero
        i2[0] = zero
        i3[0] = zero

    qv = q_ref[0]
    kv = k_ref[...]
    ab = lax.dot_general(qv, kv, (((1,), (1,)), ((), ())),
                         preferred_element_type=jnp.float32)
    f = (a2_ref[0] + b2_ref[...]) - 2.0 * ab

    om1 = m1[0]
    om2 = m2[0]
    om3 = m3[0]
    oi1 = i1[0]
    oi2 = i2[0]
    oi3 = i3[0]
    for h in range(BH):
        x = f[:, h * L:(h + 1) * L]
        p = j * BH + h
        c1 = x < om1
        c2 = x < om2
        c3 = x < om3
        nm1 = jnp.where(c1, x, om1)
        ni1 = jnp.where(c1, p, oi1)
        nm2 = jnp.where(c1, om1, jnp.where(c2, x, om2))
        ni2 = jnp.where(c1, oi1, jnp.where(c2, p, oi2))
        nm3 = jnp.where(c2, om2, jnp.where(c3, x, om3))
        ni3 = jnp.where(c2, oi2, jnp.where(c3, p, oi3))
        om1, om2, om3 = nm1, nm2, nm3
        oi1, oi2, oi3 = ni1, ni2, ni3
    m1[0] = om1
    m2[0] = om2
    m3[0] = om3
    i1[0] = oi1
    i2[0] = oi2
    i3[0] = oi3


_stream_call = pl.pallas_call(
    _stream_body,
    grid=(NROWT, NBLK),
    in_specs=[
        pl.BlockSpec((1, RT, D), lambda i, j: (i, 0, 0)),
        pl.BlockSpec((1, RT, 1), lambda i, j: (i, 0, 0)),
        pl.BlockSpec((B, D), lambda i, j: (j, 0)),
        pl.BlockSpec((1, B), lambda i, j: (0, j)),
    ],
    out_specs=[pl.BlockSpec((1, RT, L), lambda i, j: (i, 0, 0))
               for _ in range(6)],
    out_shape=[jax.ShapeDtypeStruct((NROWT, RT, L), jnp.float32)
               for _ in range(3)] +
              [jax.ShapeDtypeStruct((NROWT, RT, L), jnp.int32)
               for _ in range(3)],
    compiler_params=pltpu.CompilerParams(
        dimension_semantics=("arbitrary", "arbitrary"),
    ),
)


def _extract_body(m1_ref, m2_ref, m3_ref, i1_ref, i2_ref, i3_ref,
                  ti_ref, w_ref):
    d1 = jnp.sqrt(jnp.maximum(m1_ref[0], 1e-12))
    d2 = jnp.sqrt(jnp.maximum(m2_ref[0], 1e-12))
    d3 = jnp.sqrt(jnp.maximum(m3_ref[0], 1e-12))
    lane = lax.broadcasted_iota(jnp.int32, (RTE, L), 1)
    j1 = i1_ref[0] * L + lane
    j2 = i2_ref[0] * L + lane
    j3 = i3_ref[0] * L + lane
    bigi = jnp.int32(2**30)
    inf = jnp.float32(jnp.inf)
    top_d = []
    top_i = []
    for _ in range(KNB):
        rm = jnp.min(d1, axis=1, keepdims=True)
        si = jnp.min(jnp.where(d1 == rm, j1, bigi), axis=1, keepdims=True)
        sel = (d1 == rm) & (j1 == si)
        top_d.append(rm)
        top_i.append(si)
        d1 = jnp.where(sel, d2, d1)
        j1 = jnp.where(sel, j2, j1)
        d2 = jnp.where(sel, d3, d2)
        j2 = jnp.where(sel, j3, j2)
        d3 = jnp.where(sel, inf, d3)
    td = jnp.concatenate(top_d, axis=1)
    ti = jnp.concatenate(top_i, axis=1)
    e = jnp.exp(td[:, 0:1] - td)
    w = e / jnp.sum(e, axis=1, keepdims=True)
    ti_ref[...] = ti
    w_ref[...] = w


_extract_call = pl.pallas_call(
    _extract_body,
    grid=(NEXT,),
    in_specs=[pl.BlockSpec((1, RTE, L), lambda e: (e // (RT // RTE), e % (RT // RTE), 0))
              for _ in range(6)],
    out_specs=[
        pl.BlockSpec((RTE, KNB), lambda e: (e, 0)),
        pl.BlockSpec((RTE, KNB), lambda e: (e, 0)),
    ],
    out_shape=[
        jax.ShapeDtypeStruct((Q, KNB), jnp.int32),
        jax.ShapeDtypeStruct((Q, KNB), jnp.float32),
    ],
    compiler_params=pltpu.CompilerParams(
        dimension_semantics=("arbitrary",),
    ),
)


NW = 32
QPW = Q // NW
WPW = QPW * KNB * 8
GCH = 16
GCW = WPW // GCH
NV = QPW * 8 // 16


def _sc_blend_body(idx_hbm, w_hbm, act_hbm, out_hbm, idx_v, w_v, rows_v,
                   acc_v, sem):
    wid = lax.axis_index("s") * 2 + lax.axis_index("c")
    pltpu.sync_copy(idx_hbm.at[wid], idx_v)
    pltpu.sync_copy(w_hbm.at[wid], w_v)
    copies = [
        pltpu.async_copy(act_hbm.at[idx_v.at[c]],
                         rows_v.at[pl.ds(c * GCW, GCW)], sem)
        for c in range(GCH)
    ]
    for cp in copies:
        cp.wait()
    for v in range(NV):
        acc = w_v[0, pl.ds(v * 16, 16)] * rows_v[pl.ds(v * 16, 16)]
        for k in range(1, KNB):
            acc = acc + (w_v[k, pl.ds(v * 16, 16)] *
                         rows_v[pl.ds(k * QPW * 8 + v * 16, 16)])
        acc_v[pl.ds(v * 16, 16)] = acc
    pltpu.sync_copy(acc_v, out_hbm.at[wid])


_sc_blend_call = functools.partial(
    pl.kernel,
    out_type=jax.ShapeDtypeStruct((NW, QPW * 8), jnp.float32),
    mesh=plsc.VectorSubcoreMesh(core_axis_name="c", subcore_axis_name="s"),
    scratch_types=[
        pltpu.VMEM((GCH, GCW), jnp.int32),
        pltpu.VMEM((KNB, QPW * 8), jnp.float32),
        pltpu.VMEM((WPW,), jnp.float32),
        pltpu.VMEM((QPW * 8,), jnp.float32),
        pltpu.SemaphoreType.DMA,
    ],
    compiler_params=pltpu.CompilerParams(use_tc_tiling_on_sc=False),
)(_sc_blend_body)


def kernel(batch_images, representations, actions):
    qm = jnp.squeeze(batch_images, axis=1)
    a2 = jnp.sum(qm * qm, axis=1, keepdims=True)
    b2 = jnp.sum(representations * representations, axis=1)
    b2pad = jnp.pad(b2, (0, NPADB - N), constant_values=jnp.inf)[None, :]
    q3 = qm.reshape(NROWT, RT, D)
    a23 = a2.reshape(NROWT, RT, 1)
    m1, m2, m3, i1, i2, i3 = _stream_call(q3, a23, representations, b2pad)

    ti, w = _extract_call(m1, m2, m3, i1, i2, i3)

    af = actions.T.reshape(ADIM * N)
    toff = jnp.minimum(jnp.arange(8, dtype=jnp.int32), ADIM - 1) * N
    ie = ti[:, :, None] + toff
    idx_g = (ie.reshape(NW, QPW, KNB, 8)
             .transpose(0, 2, 1, 3).reshape(NW, GCH, GCW))
    w_g = jnp.broadcast_to(
        w.reshape(NW, QPW, KNB).transpose(0, 2, 1)[..., None],
        (NW, KNB, QPW, 8)).reshape(NW, KNB, QPW * 8)
    pred_w = _sc_blend_call(idx_g, w_g, af)
    return pred_w.reshape(Q, 8)[:, :ADIM], ti
